# baseline (speedup 1.0000x reference)
import functools

import jax
import jax.numpy as jnp
from jax import lax
from jax.experimental import pallas as pl
from jax.experimental.pallas import tpu as pltpu
from jax.experimental.pallas import tpu_sc as plsc

N = 10000
E = 320000
D = 128
G = 64
OUT = 10

NP = 10240
NW = 32
BLK = 128
BPW = 80
KG = 4
EP = NW * BPW * BLK
STRIPE = NP // 16

F1 = 32
FT = 40
F2 = 16

_f32 = jnp.float32


def _tc1_body(x_ref, w_ref, xlw_ref, xr_ref):
    xw = jnp.dot(x_ref[...], w_ref[...], preferred_element_type=_f32)
    rowid = (lax.broadcasted_iota(jnp.int32, (1024, 2 * F1), 0)
             + pl.program_id(0) * 1024)
    xw = jnp.where(rowid < N, xw, 0.0)
    xlw_ref[...] = jnp.concatenate(
        [xw[:, :F1], jnp.ones((1024, 1), _f32),
         jnp.zeros((1024, FT - F1 - 1), _f32)], axis=1)
    xr_ref[...] = xw[:, F1:]


def _tc1(x, w1):
    grid = NP // 1024
    return pl.pallas_call(
        _tc1_body,
        grid=(grid,),
        in_specs=[
            pl.BlockSpec((1024, D), lambda i: (i, 0)),
            pl.BlockSpec((D, 2 * F1), lambda i: (0, 0)),
        ],
        out_specs=[
            pl.BlockSpec((1024, FT), lambda i: (i, 0)),
            pl.BlockSpec((1024, F1), lambda i: (i, 0)),
        ],
        out_shape=[
            jax.ShapeDtypeStruct((NP, FT), _f32),
            jax.ShapeDtypeStruct((NP, F1), _f32),
        ],
    )(x, w1)


def _sc_agg_body(F, *refs):
    (table_hbm, src_hbm, dst_hbm, zf_hbm,
     sum_out,
     src_v, dst_v, rows_v, acc,
     sem_g0, sem_g1, sem_s) = refs

    c = lax.axis_index("c")
    s = lax.axis_index("s")
    wid = c * 16 + s

    pltpu.sync_copy(src_hbm.at[wid], src_v)
    pltpu.sync_copy(dst_hbm.at[wid], dst_v)
    pltpu.sync_copy(zf_hbm, acc.at[pl.ds(s * STRIPE, STRIPE)])
    plsc.subcore_barrier()

    def fire_g(t, p, sem):
        for j in range(KG):
            pltpu.async_copy(
                table_hbm.at[src_v.at[t * KG + j]], rows_v.at[p, j], sem)

    def drain_g(p, sem):
        for j in range(KG):
            pltpu.make_async_copy(
                table_hbm.at[src_v.at[j]], rows_v.at[p, j], sem).wait()

    def scatter_group(t, p):
        handles = []
        for j in range(KG):
            handles.append(pltpu.async_copy(
                rows_v.at[p, j], acc.at[dst_v.at[t * KG + j]], sem_s,
                add=True))
        for h in handles:
            h.wait()

    nu = BPW // (2 * KG)
    fire_g(0, 0, sem_g0)

    def body(u, carry):
        t0 = 2 * u
        t1 = 2 * u + 1
        drain_g(0, sem_g0)
        fire_g(t1, 1, sem_g1)
        scatter_group(t0, 0)

        @pl.when(u + 1 < nu)
        def _prefetch():
            fire_g(t0 + 2, 0, sem_g0)

        drain_g(1, sem_g1)
        scatter_group(t1, 1)
        return carry

    lax.fori_loop(0, nu, body, 0)
    plsc.subcore_barrier()

    sl = pl.ds(s * STRIPE, STRIPE)
    pltpu.sync_copy(acc.at[sl], sum_out.at[c].at[sl])


def _sc_agg(table, src_p, dst_p, F):
    mesh = plsc.VectorSubcoreMesh(
        core_axis_name="c", subcore_axis_name="s", num_cores=2, num_subcores=16)
    kern = pl.kernel(
        functools.partial(_sc_agg_body, F),
        out_type=jax.ShapeDtypeStruct((2, NP, F), _f32),
        mesh=mesh,
        scratch_types=[
            pltpu.VMEM((BPW, BLK), jnp.int32),
            pltpu.VMEM((BPW, BLK), jnp.int32),
            pltpu.VMEM((2, KG, BLK, F), _f32),
            pltpu.VMEM_SHARED((NP, F), _f32),
            pltpu.SemaphoreType.DMA,
            pltpu.SemaphoreType.DMA,
            pltpu.SemaphoreType.DMA,
        ],
        compiler_params=pltpu.CompilerParams(use_tc_tiling_on_sc=False),
    )
    return kern(table, src_p, dst_p, jnp.zeros((STRIPE, F), _f32))


def _tc2_body(s1_ref, xr_ref, b1_ref, w2_ref, hl_ref, hr_ref, rc_ref):
    ssum = s1_ref[0] + s1_ref[1]
    r = 1.0 / jnp.maximum(ssum[:, F1:F1 + 1], 1.0)
    h = jnp.maximum(ssum[:, :F1] * r + xr_ref[...] + b1_ref[...], 0.0)
    hw = jnp.dot(h, w2_ref[...], preferred_element_type=_f32)
    hl_ref[...] = hw[:, :F2]
    hr_ref[...] = hw[:, F2:]
    rc_ref[...] = jnp.broadcast_to(r, (1024, 8))


def _tc2(s1, xr, b1, w2):
    grid = NP // 1024
    return pl.pallas_call(
        _tc2_body,
        grid=(grid,),
        in_specs=[
            pl.BlockSpec((2, 1024, FT), lambda i: (0, i, 0)),
            pl.BlockSpec((1024, F1), lambda i: (i, 0)),
            pl.BlockSpec((1, F1), lambda i: (0, 0)),
            pl.BlockSpec((F1, 2 * F2), lambda i: (0, 0)),
        ],
        out_specs=[
            pl.BlockSpec((1024, F2), lambda i: (i, 0)),
            pl.BlockSpec((1024, F2), lambda i: (i, 0)),
            pl.BlockSpec((1024, 8), lambda i: (i, 0)),
        ],
        out_shape=[
            jax.ShapeDtypeStruct((NP, F2), _f32),
            jax.ShapeDtypeStruct((NP, F2), _f32),
            jax.ShapeDtypeStruct((NP, 8), _f32),
        ],
    )(s1, xr, b1, w2)


def _tc3_body(s2_ref, rc_ref, hr_ref, b2_ref, batch_ref, wlin_ref, blin_ref,
              out_ref, acc_ref):
    i = pl.program_id(0)

    @pl.when(i == 0)
    def _init():
        acc_ref[...] = jnp.zeros_like(acc_ref)

    ssum = s2_ref[0] + s2_ref[1]
    h2 = ssum * rc_ref[:, 0:1] + hr_ref[...] + b2_ref[...]
    bvec = batch_ref[0]
    onehot = (bvec[None, :] == lax.broadcasted_iota(jnp.int32, (G, 1024), 0)
              ).astype(_f32)
    h2p = jnp.concatenate(
        [h2, jnp.ones((1024, 1), _f32), jnp.zeros((1024, F2 - 1), _f32)],
        axis=1)
    acc_ref[...] += jnp.dot(onehot, h2p, preferred_element_type=_f32)

    @pl.when(i == (NP // 1024) - 1)
    def _fin():
        a = acc_ref[...]
        pooled = a[:, :F2] / jnp.maximum(a[:, F2:F2 + 1], 1.0)
        out_ref[...] = (jnp.dot(pooled, wlin_ref[...],
                                preferred_element_type=_f32) + blin_ref[...])


def _tc3(s2, rc, hr, b2, batch_p, wlin, blin):
    grid = NP // 1024
    return pl.pallas_call(
        _tc3_body,
        grid=(grid,),
        in_specs=[
            pl.BlockSpec((2, 1024, F2), lambda i: (0, i, 0)),
            pl.BlockSpec((1024, 8), lambda i: (i, 0)),
            pl.BlockSpec((1024, F2), lambda i: (i, 0)),
            pl.BlockSpec((1, F2), lambda i: (0, 0)),
            pl.BlockSpec((1, 1024), lambda i: (0, i)),
            pl.BlockSpec((F2, OUT), lambda i: (0, 0)),
            pl.BlockSpec((1, OUT), lambda i: (0, 0)),
        ],
        out_specs=pl.BlockSpec((G, OUT), lambda i: (0, 0)),
        out_shape=jax.ShapeDtypeStruct((G, OUT), _f32),
        scratch_shapes=[pltpu.VMEM((G, 2 * F2), _f32)],
    )(s2, rc, hr, b2, batch_p, wlin, blin)


def kernel(x, edge_index, batch, w1l, w1r, b1, w2l, w2r, b2, wlin, blin):
    src = edge_index[0]
    dst = edge_index[1]
    src_p = jnp.concatenate(
        [src, jnp.zeros((EP - E,), jnp.int32)]).reshape(NW, BPW, BLK)
    dst_p = jnp.concatenate(
        [dst, jnp.full((EP - E,), N, jnp.int32)]).reshape(NW, BPW, BLK)
    batch_p = jnp.concatenate(
        [batch, jnp.full((NP - N,), G, jnp.int32)]).reshape(1, NP)

    w1 = jnp.concatenate([w1l, w1r], axis=1)
    w2 = jnp.concatenate([w2l, w2r], axis=1)

    xlw, xr = _tc1(x, w1)
    s1 = _sc_agg(xlw, src_p, dst_p, FT)
    hl, hr, rc = _tc2(s1, xr, b1.reshape(1, F1), w2)
    s2 = _sc_agg(hl, src_p, dst_p, F2)
    return _tc3(s2, rc, hr, b2.reshape(1, F2), batch_p, wlin,
                blin.reshape(1, OUT))


# speedup vs baseline: 1.0813x; 1.0813x over previous
import functools

import jax
import jax.numpy as jnp
from jax import lax
from jax.experimental import pallas as pl
from jax.experimental.pallas import tpu as pltpu
from jax.experimental.pallas import tpu_sc as plsc

N = 10000
E = 320000
D = 128
G = 64
OUT = 10

NP = 10240
NW = 32
BLK = 128
BPW = 80
KG = 4
EP = NW * BPW * BLK
STRIPE = NP // 16

F1 = 32
F2 = 16
CW = 8

_f32 = jnp.float32


def _tc1_body(x_ref, w_ref, xl_ref, xr_ref):
    xw = jnp.dot(x_ref[...], w_ref[...], preferred_element_type=_f32)
    xl_ref[...] = xw[:, :F1]
    xr_ref[...] = xw[:, F1:]


def _tc1(x_p, w1):
    grid = NP // 1024
    return pl.pallas_call(
        _tc1_body,
        grid=(grid,),
        in_specs=[
            pl.BlockSpec((1024, D), lambda i: (i, 0)),
            pl.BlockSpec((D, 2 * F1), lambda i: (0, 0)),
        ],
        out_specs=[
            pl.BlockSpec((1024, F1), lambda i: (i, 0)),
            pl.BlockSpec((1024, F1), lambda i: (i, 0)),
        ],
        out_shape=[
            jax.ShapeDtypeStruct((NP, F1), _f32),
            jax.ShapeDtypeStruct((NP, F1), _f32),
        ],
    )(x_p, w1)


def _sc_agg_body(F, with_cnt, *refs):
    if with_cnt:
        (table_hbm, src_hbm, dst_hbm, zf_hbm, zc_hbm, ones_hbm,
         sum_out, cnt_out,
         src_v, dst_v, rows_v, ones_v, acc, accc,
         sem_g0, sem_g1, sem_s) = refs
    else:
        (table_hbm, src_hbm, dst_hbm, zf_hbm,
         sum_out,
         src_v, dst_v, rows_v, acc,
         sem_g0, sem_g1, sem_s) = refs
        ones_hbm = ones_v = accc = None

    c = lax.axis_index("c")
    s = lax.axis_index("s")
    wid = c * 16 + s

    pltpu.sync_copy(src_hbm.at[wid], src_v)
    pltpu.sync_copy(dst_hbm.at[wid], dst_v)
    pltpu.sync_copy(zf_hbm, acc.at[pl.ds(s * STRIPE, STRIPE)])
    if with_cnt:
        pltpu.sync_copy(zc_hbm, accc.at[pl.ds(s * STRIPE, STRIPE)])
        pltpu.sync_copy(ones_hbm, ones_v)
    plsc.subcore_barrier()

    def fire_g(t, p, sem):
        for j in range(KG):
            pltpu.async_copy(
                table_hbm.at[src_v.at[t * KG + j]], rows_v.at[p, j], sem)

    def drain_g(p, sem):
        for j in range(KG):
            pltpu.make_async_copy(
                table_hbm.at[src_v.at[j]], rows_v.at[p, j], sem).wait()

    def scatter_group(t, p):
        handles = []
        for j in range(KG):
            blk = t * KG + j
            handles.append(pltpu.async_copy(
                rows_v.at[p, j], acc.at[dst_v.at[blk]], sem_s, add=True))
            if with_cnt:
                handles.append(pltpu.async_copy(
                    ones_v, accc.at[dst_v.at[blk]], sem_s, add=True))
        for h in handles:
            h.wait()

    nu = BPW // (2 * KG)
    fire_g(0, 0, sem_g0)

    def body(u, carry):
        t0 = 2 * u
        t1 = 2 * u + 1
        drain_g(0, sem_g0)
        fire_g(t1, 1, sem_g1)
        scatter_group(t0, 0)

        @pl.when(u + 1 < nu)
        def _prefetch():
            fire_g(t0 + 2, 0, sem_g0)

        drain_g(1, sem_g1)
        scatter_group(t1, 1)
        return carry

    lax.fori_loop(0, nu, body, 0)
    plsc.subcore_barrier()

    sl = pl.ds(s * STRIPE, STRIPE)
    pltpu.sync_copy(acc.at[sl], sum_out.at[c].at[sl])
    if with_cnt:
        pltpu.sync_copy(accc.at[sl], cnt_out.at[c].at[sl])


def _sc_agg(table, src_p, dst_p, F, with_cnt):
    mesh = plsc.VectorSubcoreMesh(
        core_axis_name="c", subcore_axis_name="s", num_cores=2, num_subcores=16)
    zf = jnp.zeros((STRIPE, F), _f32)
    if with_cnt:
        out_type = [jax.ShapeDtypeStruct((2, NP, F), _f32),
                    jax.ShapeDtypeStruct((2, NP, CW), _f32)]
        extra_in = (zf, jnp.zeros((STRIPE, CW), _f32), jnp.ones((BLK, CW), _f32))
        extra_scratch = [pltpu.VMEM((BLK, CW), _f32),
                         pltpu.VMEM_SHARED((NP, F), _f32),
                         pltpu.VMEM_SHARED((NP, CW), _f32)]
    else:
        out_type = jax.ShapeDtypeStruct((2, NP, F), _f32)
        extra_in = (zf,)
        extra_scratch = [pltpu.VMEM_SHARED((NP, F), _f32)]

    kern = pl.kernel(
        functools.partial(_sc_agg_body, F, with_cnt),
        out_type=out_type,
        mesh=mesh,
        scratch_types=[
            pltpu.VMEM((BPW, BLK), jnp.int32),
            pltpu.VMEM((BPW, BLK), jnp.int32),
            pltpu.VMEM((2, KG, BLK, F), _f32),
        ] + extra_scratch + [pltpu.SemaphoreType.DMA,
                             pltpu.SemaphoreType.DMA,
                             pltpu.SemaphoreType.DMA],
        compiler_params=pltpu.CompilerParams(use_tc_tiling_on_sc=False),
    )
    return kern(table, src_p, dst_p, *extra_in)


def _tc2_body(s1_ref, cnt_ref, xr_ref, b1_ref, w2_ref, hl_ref, hr_ref):
    ssum = s1_ref[0] + s1_ref[1]
    csum = cnt_ref[0] + cnt_ref[1]
    r = 1.0 / jnp.maximum(csum[:, 0:1], 1.0)
    h = jnp.maximum(ssum * r + xr_ref[...] + b1_ref[...], 0.0)
    hw = jnp.dot(h, w2_ref[...], preferred_element_type=_f32)
    hl_ref[...] = hw[:, :F2]
    hr_ref[...] = hw[:, F2:]


def _tc2(s1, cnt, xr, b1, w2):
    grid = NP // 1024
    return pl.pallas_call(
        _tc2_body,
        grid=(grid,),
        in_specs=[
            pl.BlockSpec((2, 1024, F1), lambda i: (0, i, 0)),
            pl.BlockSpec((2, 1024, CW), lambda i: (0, i, 0)),
            pl.BlockSpec((1024, F1), lambda i: (i, 0)),
            pl.BlockSpec((1, F1), lambda i: (0, 0)),
            pl.BlockSpec((F1, 2 * F2), lambda i: (0, 0)),
        ],
        out_specs=[
            pl.BlockSpec((1024, F2), lambda i: (i, 0)),
            pl.BlockSpec((1024, F2), lambda i: (i, 0)),
        ],
        out_shape=[
            jax.ShapeDtypeStruct((NP, F2), _f32),
            jax.ShapeDtypeStruct((NP, F2), _f32),
        ],
    )(s1, cnt, xr, b1, w2)


def _tc3_body(s2_ref, cnt_ref, hr_ref, b2_ref, batch_ref, wlin_ref, blin_ref,
              out_ref, acc_ref):
    i = pl.program_id(0)

    @pl.when(i == 0)
    def _init():
        acc_ref[...] = jnp.zeros_like(acc_ref)

    ssum = s2_ref[0] + s2_ref[1]
    csum = cnt_ref[0] + cnt_ref[1]
    r = 1.0 / jnp.maximum(csum[:, 0:1], 1.0)
    h2 = ssum * r + hr_ref[...] + b2_ref[...]
    bvec = batch_ref[0]
    onehot = (bvec[None, :] == lax.broadcasted_iota(jnp.int32, (G, 1024), 0)
              ).astype(_f32)
    h2p = jnp.concatenate(
        [h2, jnp.ones((1024, 1), _f32), jnp.zeros((1024, F2 - 1), _f32)],
        axis=1)
    acc_ref[...] += jnp.dot(onehot, h2p, preferred_element_type=_f32)

    @pl.when(i == (NP // 1024) - 1)
    def _fin():
        a = acc_ref[...]
        pooled = a[:, :F2] / jnp.maximum(a[:, F2:F2 + 1], 1.0)
        out_ref[...] = (jnp.dot(pooled, wlin_ref[...],
                                preferred_element_type=_f32) + blin_ref[...])


def _tc3(s2, cnt, hr, b2, batch_p, wlin, blin):
    grid = NP // 1024
    return pl.pallas_call(
        _tc3_body,
        grid=(grid,),
        in_specs=[
            pl.BlockSpec((2, 1024, F2), lambda i: (0, i, 0)),
            pl.BlockSpec((2, 1024, CW), lambda i: (0, i, 0)),
            pl.BlockSpec((1024, F2), lambda i: (i, 0)),
            pl.BlockSpec((1, F2), lambda i: (0, 0)),
            pl.BlockSpec((1, 1024), lambda i: (0, i)),
            pl.BlockSpec((F2, OUT), lambda i: (0, 0)),
            pl.BlockSpec((1, OUT), lambda i: (0, 0)),
        ],
        out_specs=pl.BlockSpec((G, OUT), lambda i: (0, 0)),
        out_shape=jax.ShapeDtypeStruct((G, OUT), _f32),
        scratch_shapes=[pltpu.VMEM((G, 2 * F2), _f32)],
    )(s2, cnt, hr, b2, batch_p, wlin, blin)


def kernel(x, edge_index, batch, w1l, w1r, b1, w2l, w2r, b2, wlin, blin):
    src = edge_index[0]
    dst = edge_index[1]
    src_p = jnp.concatenate(
        [src, jnp.zeros((EP - E,), jnp.int32)]).reshape(NW, BPW, BLK)
    dst_p = jnp.concatenate(
        [dst, jnp.full((EP - E,), N, jnp.int32)]).reshape(NW, BPW, BLK)
    x_p = jnp.concatenate([x, jnp.zeros((NP - N, D), _f32)], axis=0)
    batch_p = jnp.concatenate(
        [batch, jnp.full((NP - N,), G, jnp.int32)]).reshape(1, NP)

    w1 = jnp.concatenate([w1l, w1r], axis=1)
    w2 = jnp.concatenate([w2l, w2r], axis=1)

    xl, xr = _tc1(x_p, w1)
    s1, cnt = _sc_agg(xl, src_p, dst_p, F1, True)
    hl, hr = _tc2(s1, cnt, xr, b1.reshape(1, F1), w2)
    s2 = _sc_agg(hl, src_p, dst_p, F2, False)
    return _tc3(s2, cnt, hr, b2.reshape(1, F2), batch_p, wlin,
                blin.reshape(1, OUT))


# speedup vs baseline: 1.6565x; 1.5320x over previous
import functools

import jax
import jax.numpy as jnp
from jax import lax
from jax.experimental import pallas as pl
from jax.experimental.pallas import tpu as pltpu
from jax.experimental.pallas import tpu_sc as plsc

N = 10000
E = 320000
D = 128
G = 64
OUT = 10

NP = 10240
NW = 32
BLK = 128
BPW = 80
KG = 4
EP = NW * BPW * BLK
STRIPE = NP // 16

F1 = 32
F2 = 16
CW = 8

_f32 = jnp.float32


def _tc1_body(x_ref, w_ref, xl_ref, xr_ref):
    xw = jnp.dot(x_ref[...], w_ref[...], preferred_element_type=_f32)
    xl_ref[...] = xw[:, :F1]
    xr_ref[...] = xw[:, F1:]


def _tc1(x_p, w1):
    grid = NP // 1024
    return pl.pallas_call(
        _tc1_body,
        grid=(grid,),
        in_specs=[
            pl.BlockSpec((1024, D), lambda i: (i, 0)),
            pl.BlockSpec((D, 2 * F1), lambda i: (0, 0)),
        ],
        out_specs=[
            pl.BlockSpec((1024, F1), lambda i: (i, 0)),
            pl.BlockSpec((1024, F1), lambda i: (i, 0)),
        ],
        out_shape=[
            jax.ShapeDtypeStruct((NP, F1), _f32),
            jax.ShapeDtypeStruct((NP, F1), _f32),
        ],
    )(x_p, w1)


def _sc_agg_body(F, with_cnt, *refs):
    if with_cnt:
        (table_hbm, src_hbm, dst_hbm, zf_hbm, zc_hbm, ones_hbm,
         sum_out, cnt_out,
         src_v, dst_v, rows_v, ones_v, acc, accc, tbl,
         sem_g0, sem_g1, sem_s) = refs
    else:
        (table_hbm, src_hbm, dst_hbm, zf_hbm,
         sum_out,
         src_v, dst_v, rows_v, acc, tbl,
         sem_g0, sem_g1, sem_s) = refs
        ones_hbm = ones_v = accc = None

    c = lax.axis_index("c")
    s = lax.axis_index("s")
    wid = c * 16 + s

    pltpu.sync_copy(src_hbm.at[wid], src_v)
    pltpu.sync_copy(dst_hbm.at[wid], dst_v)
    pltpu.sync_copy(zf_hbm, acc.at[pl.ds(s * STRIPE, STRIPE)])
    pltpu.sync_copy(table_hbm.at[pl.ds(s * STRIPE, STRIPE)],
                    tbl.at[pl.ds(s * STRIPE, STRIPE)])
    if with_cnt:
        pltpu.sync_copy(zc_hbm, accc.at[pl.ds(s * STRIPE, STRIPE)])
        pltpu.sync_copy(ones_hbm, ones_v)
    plsc.subcore_barrier()

    def fire_g(t, p, sem):
        for j in range(KG):
            pltpu.async_copy(
                tbl.at[src_v.at[t * KG + j]], rows_v.at[p, j], sem)

    def drain_g(p, sem):
        for j in range(KG):
            pltpu.make_async_copy(
                tbl.at[src_v.at[j]], rows_v.at[p, j], sem).wait()

    def scatter_group(t, p):
        handles = []
        for j in range(KG):
            blk = t * KG + j
            handles.append(pltpu.async_copy(
                rows_v.at[p, j], acc.at[dst_v.at[blk]], sem_s, add=True))
            if with_cnt:
                handles.append(pltpu.async_copy(
                    ones_v, accc.at[dst_v.at[blk]], sem_s, add=True))
        for h in handles:
            h.wait()

    nu = BPW // (2 * KG)
    fire_g(0, 0, sem_g0)

    def body(u, carry):
        t0 = 2 * u
        t1 = 2 * u + 1
        drain_g(0, sem_g0)
        fire_g(t1, 1, sem_g1)
        scatter_group(t0, 0)

        @pl.when(u + 1 < nu)
        def _prefetch():
            fire_g(t0 + 2, 0, sem_g0)

        drain_g(1, sem_g1)
        scatter_group(t1, 1)
        return carry

    lax.fori_loop(0, nu, body, 0)
    plsc.subcore_barrier()

    sl = pl.ds(s * STRIPE, STRIPE)
    pltpu.sync_copy(acc.at[sl], sum_out.at[c].at[sl])
    if with_cnt:
        pltpu.sync_copy(accc.at[sl], cnt_out.at[c].at[sl])


def _sc_agg(table, src_p, dst_p, F, with_cnt):
    mesh = plsc.VectorSubcoreMesh(
        core_axis_name="c", subcore_axis_name="s", num_cores=2, num_subcores=16)
    zf = jnp.zeros((STRIPE, F), _f32)
    if with_cnt:
        out_type = [jax.ShapeDtypeStruct((2, NP, F), _f32),
                    jax.ShapeDtypeStruct((2, NP, CW), _f32)]
        extra_in = (zf, jnp.zeros((STRIPE, CW), _f32), jnp.ones((BLK, CW), _f32))
        extra_scratch = [pltpu.VMEM((BLK, CW), _f32),
                         pltpu.VMEM_SHARED((NP, F), _f32),
                         pltpu.VMEM_SHARED((NP, CW), _f32),
                         pltpu.VMEM_SHARED((NP, F), _f32)]
    else:
        out_type = jax.ShapeDtypeStruct((2, NP, F), _f32)
        extra_in = (zf,)
        extra_scratch = [pltpu.VMEM_SHARED((NP, F), _f32),
                         pltpu.VMEM_SHARED((NP, F), _f32)]

    kern = pl.kernel(
        functools.partial(_sc_agg_body, F, with_cnt),
        out_type=out_type,
        mesh=mesh,
        scratch_types=[
            pltpu.VMEM((BPW, BLK), jnp.int32),
            pltpu.VMEM((BPW, BLK), jnp.int32),
            pltpu.VMEM((2, KG, BLK, F), _f32),
        ] + extra_scratch + [pltpu.SemaphoreType.DMA,
                             pltpu.SemaphoreType.DMA,
                             pltpu.SemaphoreType.DMA],
        compiler_params=pltpu.CompilerParams(use_tc_tiling_on_sc=False),
    )
    return kern(table, src_p, dst_p, *extra_in)


def _tc2_body(s1_ref, cnt_ref, xr_ref, b1_ref, w2_ref, hl_ref, hr_ref):
    ssum = s1_ref[0] + s1_ref[1]
    csum = cnt_ref[0] + cnt_ref[1]
    r = 1.0 / jnp.maximum(csum[:, 0:1], 1.0)
    h = jnp.maximum(ssum * r + xr_ref[...] + b1_ref[...], 0.0)
    hw = jnp.dot(h, w2_ref[...], preferred_element_type=_f32)
    hl_ref[...] = hw[:, :F2]
    hr_ref[...] = hw[:, F2:]


def _tc2(s1, cnt, xr, b1, w2):
    grid = NP // 1024
    return pl.pallas_call(
        _tc2_body,
        grid=(grid,),
        in_specs=[
            pl.BlockSpec((2, 1024, F1), lambda i: (0, i, 0)),
            pl.BlockSpec((2, 1024, CW), lambda i: (0, i, 0)),
            pl.BlockSpec((1024, F1), lambda i: (i, 0)),
            pl.BlockSpec((1, F1), lambda i: (0, 0)),
            pl.BlockSpec((F1, 2 * F2), lambda i: (0, 0)),
        ],
        out_specs=[
            pl.BlockSpec((1024, F2), lambda i: (i, 0)),
            pl.BlockSpec((1024, F2), lambda i: (i, 0)),
        ],
        out_shape=[
            jax.ShapeDtypeStruct((NP, F2), _f32),
            jax.ShapeDtypeStruct((NP, F2), _f32),
        ],
    )(s1, cnt, xr, b1, w2)


def _tc3_body(s2_ref, cnt_ref, hr_ref, b2_ref, batch_ref, wlin_ref, blin_ref,
              out_ref, acc_ref):
    i = pl.program_id(0)

    @pl.when(i == 0)
    def _init():
        acc_ref[...] = jnp.zeros_like(acc_ref)

    ssum = s2_ref[0] + s2_ref[1]
    csum = cnt_ref[0] + cnt_ref[1]
    r = 1.0 / jnp.maximum(csum[:, 0:1], 1.0)
    h2 = ssum * r + hr_ref[...] + b2_ref[...]
    bvec = batch_ref[0]
    onehot = (bvec[None, :] == lax.broadcasted_iota(jnp.int32, (G, 1024), 0)
              ).astype(_f32)
    h2p = jnp.concatenate(
        [h2, jnp.ones((1024, 1), _f32), jnp.zeros((1024, F2 - 1), _f32)],
        axis=1)
    acc_ref[...] += jnp.dot(onehot, h2p, preferred_element_type=_f32)

    @pl.when(i == (NP // 1024) - 1)
    def _fin():
        a = acc_ref[...]
        pooled = a[:, :F2] / jnp.maximum(a[:, F2:F2 + 1], 1.0)
        out_ref[...] = (jnp.dot(pooled, wlin_ref[...],
                                preferred_element_type=_f32) + blin_ref[...])


def _tc3(s2, cnt, hr, b2, batch_p, wlin, blin):
    grid = NP // 1024
    return pl.pallas_call(
        _tc3_body,
        grid=(grid,),
        in_specs=[
            pl.BlockSpec((2, 1024, F2), lambda i: (0, i, 0)),
            pl.BlockSpec((2, 1024, CW), lambda i: (0, i, 0)),
            pl.BlockSpec((1024, F2), lambda i: (i, 0)),
            pl.BlockSpec((1, F2), lambda i: (0, 0)),
            pl.BlockSpec((1, 1024), lambda i: (0, i)),
            pl.BlockSpec((F2, OUT), lambda i: (0, 0)),
            pl.BlockSpec((1, OUT), lambda i: (0, 0)),
        ],
        out_specs=pl.BlockSpec((G, OUT), lambda i: (0, 0)),
        out_shape=jax.ShapeDtypeStruct((G, OUT), _f32),
        scratch_shapes=[pltpu.VMEM((G, 2 * F2), _f32)],
    )(s2, cnt, hr, b2, batch_p, wlin, blin)


def kernel(x, edge_index, batch, w1l, w1r, b1, w2l, w2r, b2, wlin, blin):
    src = edge_index[0]
    dst = edge_index[1]
    src_p = jnp.concatenate(
        [src, jnp.zeros((EP - E,), jnp.int32)]).reshape(NW, BPW, BLK)
    dst_p = jnp.concatenate(
        [dst, jnp.full((EP - E,), N, jnp.int32)]).reshape(NW, BPW, BLK)
    x_p = jnp.concatenate([x, jnp.zeros((NP - N, D), _f32)], axis=0)
    batch_p = jnp.concatenate(
        [batch, jnp.full((NP - N,), G, jnp.int32)]).reshape(1, NP)

    w1 = jnp.concatenate([w1l, w1r], axis=1)
    w2 = jnp.concatenate([w2l, w2r], axis=1)

    xl, xr = _tc1(x_p, w1)
    s1, cnt = _sc_agg(xl, src_p, dst_p, F1, True)
    hl, hr = _tc2(s1, cnt, xr, b1.reshape(1, F1), w2)
    s2 = _sc_agg(hl, src_p, dst_p, F2, False)
    return _tc3(s2, cnt, hr, b2.reshape(1, F2), batch_p, wlin,
                blin.reshape(1, OUT))
